# baseline (speedup 1.0000x reference)
import functools

import jax
import jax.numpy as jnp
from jax import lax
from jax.experimental import pallas as pl
from jax.experimental.pallas import tpu as pltpu
from jax.experimental.pallas import tpu_sc as plsc

_LEFT = 32
_TILE = 256
_WIN = 384
_CHUNK = 24


def _sc_gather(M, C, idx_pad):
    n_pad = idx_pad.shape[0]
    V, D = M.shape
    info = plsc.get_sparse_core_info()
    nc, ns = info.num_cores, info.num_subcores
    r1 = (n_pad // (2 * ns) + 8) // 8 * 8
    r0 = n_pad // ns - r1
    mesh = plsc.VectorSubcoreMesh(core_axis_name="c", subcore_axis_name="s")

    @functools.partial(
        pl.kernel,
        mesh=mesh,
        out_type=[
            jax.ShapeDtypeStruct((n_pad, D), jnp.float32),
            jax.ShapeDtypeStruct((n_pad, D), jnp.float32),
        ],
        scratch_types=[
            pltpu.VMEM((max(r0, r1),), jnp.int32),
            pltpu.VMEM((2, max(r0, r1), D), jnp.float32),
            pltpu.SemaphoreType.DMA,
            pltpu.SemaphoreType.DMA,
        ],
    )
    def gather_kernel(m_hbm, c_hbm, idx_hbm, outm_hbm, outc_hbm,
                      idx_v, rows_v, sem_m, sem_c):
        c = lax.axis_index("c")
        s = lax.axis_index("s")

        def work(base, n):
            idx_c = idx_v.at[pl.ds(0, n)]
            pltpu.sync_copy(idx_hbm.at[pl.ds(base, n)], idx_c)
            cp_m = pltpu.async_copy(m_hbm.at[idx_c], rows_v.at[0, pl.ds(0, n)], sem_m)
            cp_c = pltpu.async_copy(c_hbm.at[idx_c], rows_v.at[1, pl.ds(0, n)], sem_c)
            cp_m.wait()
            pltpu.sync_copy(rows_v.at[0, pl.ds(0, n)], outm_hbm.at[pl.ds(base, n)])
            cp_c.wait()
            pltpu.sync_copy(rows_v.at[1, pl.ds(0, n)], outc_hbm.at[pl.ds(base, n)])

        pl.when(c == 0)(lambda: work(s * r0, r0))
        pl.when(c != 0)(lambda: work(ns * r0 + s * r1, r1))

    return gather_kernel(M, C, idx_pad)


def _attn_body(e_ref, m0_ref, m1_ref, m2_ref, c0_ref, c1_ref, c2_ref,
               out_ref, p_ref):
    E = e_ref[0]
    Wm = jnp.concatenate([m0_ref[...], m1_ref[...], m2_ref[...]], axis=0)
    A = lax.dot_general(E, Wm, (((1,), (1,)), ((), ())),
                        preferred_element_type=jnp.float32)
    t_i = lax.broadcasted_iota(jnp.int32, (_TILE, _WIN), 0)
    c_i = lax.broadcasted_iota(jnp.int32, (_TILE, _WIN), 1)
    delta = c_i - t_i
    band = (delta >= 0) & (delta < _LEFT)
    Am = jnp.where(band, A, -1e30)
    m = jnp.max(Am, axis=1, keepdims=True)
    ex = jnp.exp(Am - m)
    denom = jnp.sum(ex, axis=1, keepdims=True)
    pn = ex / denom
    Wc = jnp.concatenate([c0_ref[...], c1_ref[...], c2_ref[...]], axis=0)
    comp = lax.dot_general(pn, Wc, (((1,), (0,)), ((), ())),
                           preferred_element_type=jnp.float32)
    D = E.shape[1]
    out_ref[0, :, :D] = comp
    out_ref[0, :, D:] = E
    x = pn
    for b in range(8):
        k = 1 << b
        rolled = jnp.concatenate([x[:, k:], x[:, :k]], axis=1)
        bit = (lax.broadcasted_iota(jnp.int32, (_TILE, _WIN), 0) >> b) & 1
        x = jnp.where(bit == 1, rolled, x)
    p_ref[0] = x[:, :_LEFT]


def _attn_tc(enc, Mrows, Crows, interpret=False):
    B, S, D = enc.shape
    n_pad = Mrows.shape[0]
    ntiles = S // _TILE
    out, p = pl.pallas_call(
        _attn_body,
        grid=(ntiles,),
        in_specs=[
            pl.BlockSpec((1, _TILE, D), lambda i: (0, i, 0)),
            pl.BlockSpec((128, D), lambda i: (2 * i, 0)),
            pl.BlockSpec((128, D), lambda i: (2 * i + 1, 0)),
            pl.BlockSpec((128, D), lambda i: (2 * i + 2, 0)),
            pl.BlockSpec((128, D), lambda i: (2 * i, 0)),
            pl.BlockSpec((128, D), lambda i: (2 * i + 1, 0)),
            pl.BlockSpec((128, D), lambda i: (2 * i + 2, 0)),
        ],
        out_specs=[
            pl.BlockSpec((1, _TILE, 2 * D), lambda i: (0, i, 0)),
            pl.BlockSpec((1, _TILE, _LEFT), lambda i: (0, i, 0)),
        ],
        out_shape=[
            jax.ShapeDtypeStruct((1, S, 2 * D), jnp.float32),
            jax.ShapeDtypeStruct((1, S, _LEFT), jnp.float32),
        ],
        interpret=interpret,
    )(enc, Mrows, Mrows, Mrows, Crows, Crows, Crows)
    return out, p


def kernel(symbols, encodings, M, C):
    B, S = symbols.shape
    D = encodings.shape[-1]
    sym = symbols[0].astype(jnp.int32)
    n_pad = ((S + _LEFT - 1) // _TILE + 1) * _TILE
    idx_pad = jnp.concatenate([
        jnp.full((_LEFT - 1,), sym[0], jnp.int32),
        sym,
        jnp.zeros((n_pad - S - (_LEFT - 1),), jnp.int32),
    ])
    Mrows, Crows = _sc_gather(M, C, idx_pad)
    return _attn_tc(encodings, Mrows, Crows)


# speedup vs baseline: 1.0782x; 1.0782x over previous
import functools

import jax
import jax.numpy as jnp
from jax import lax
from jax.experimental import pallas as pl
from jax.experimental.pallas import tpu as pltpu
from jax.experimental.pallas import tpu_sc as plsc

_LEFT = 32
_TILE = 256
_WIN = 384
_CHUNK = 24


def _sc_gather(M, C, idx_pad):
    n_pad = idx_pad.shape[0]
    V, D = M.shape
    info = plsc.get_sparse_core_info()
    nc, ns = info.num_cores, info.num_subcores
    r0 = 112
    r1 = n_pad // ns - r0
    ch = 16
    depth = 6
    lag = 3
    mesh = plsc.VectorSubcoreMesh(core_axis_name="c", subcore_axis_name="s")

    @functools.partial(
        pl.kernel,
        mesh=mesh,
        out_type=[
            jax.ShapeDtypeStruct((n_pad, D), jnp.float32),
            jax.ShapeDtypeStruct((n_pad, D), jnp.float32),
        ],
        scratch_types=[
            pltpu.VMEM((max(r0, r1),), jnp.int32),
            pltpu.VMEM((depth, ch, D), jnp.float32),
        ]
        + [pltpu.SemaphoreType.DMA] * (2 * depth),
    )
    def gather_kernel(m_hbm, c_hbm, idx_hbm, outm_hbm, outc_hbm,
                      idx_v, rows_v, *sems):
        g_sems = sems[:depth]
        w_sems = sems[depth:]
        c = lax.axis_index("c")
        s = lax.axis_index("s")

        def work(base, n):
            pltpu.sync_copy(idx_hbm.at[pl.ds(base, n)], idx_v.at[pl.ds(0, n)])
            jobs = []
            for u in range(n // ch):
                jobs.append((m_hbm, outm_hbm, u * ch))
                jobs.append((c_hbm, outc_hbm, u * ch))
            g_cp = [None] * len(jobs)
            w_cp = [None] * len(jobs)

            def writeback(j):
                src, dst, off = jobs[j]
                slot = j % depth
                g_cp[j].wait()
                w_cp[j] = pltpu.async_copy(
                    rows_v.at[slot], dst.at[pl.ds(base + off, ch)], w_sems[slot])

            for j, (src, dst, off) in enumerate(jobs):
                slot = j % depth
                if j >= depth:
                    w_cp[j - depth].wait()
                g_cp[j] = pltpu.async_copy(
                    src.at[idx_v.at[pl.ds(off, ch)]], rows_v.at[slot], g_sems[slot])
                if j >= lag:
                    writeback(j - lag)
            for j in range(max(len(jobs) - lag, 0), len(jobs)):
                writeback(j)
            for j in range(max(len(jobs) - depth, 0), len(jobs)):
                w_cp[j].wait()

        pl.when(c == 0)(lambda: work(s * r0, r0))
        pl.when(c != 0)(lambda: work(ns * r0 + s * r1, r1))

    return gather_kernel(M, C, idx_pad)


def _attn_body(e_ref, m0_ref, m1_ref, m2_ref, c0_ref, c1_ref, c2_ref,
               out_ref, p_ref):
    E = e_ref[0]
    Wm = jnp.concatenate([m0_ref[...], m1_ref[...], m2_ref[...]], axis=0)
    A = lax.dot_general(E, Wm, (((1,), (1,)), ((), ())),
                        preferred_element_type=jnp.float32)
    t_i = lax.broadcasted_iota(jnp.int32, (_TILE, _WIN), 0)
    c_i = lax.broadcasted_iota(jnp.int32, (_TILE, _WIN), 1)
    delta = c_i - t_i
    band = (delta >= 0) & (delta < _LEFT)
    Am = jnp.where(band, A, -1e30)
    m = jnp.max(Am, axis=1, keepdims=True)
    ex = jnp.exp(Am - m)
    denom = jnp.sum(ex, axis=1, keepdims=True)
    pn = ex / denom
    Wc = jnp.concatenate([c0_ref[...], c1_ref[...], c2_ref[...]], axis=0)
    comp = lax.dot_general(pn, Wc, (((1,), (0,)), ((), ())),
                           preferred_element_type=jnp.float32)
    D = E.shape[1]
    out_ref[0, :, :D] = comp
    out_ref[0, :, D:] = E
    x = pn
    for b in range(8):
        k = 1 << b
        rolled = jnp.concatenate([x[:, k:], x[:, :k]], axis=1)
        bit = (lax.broadcasted_iota(jnp.int32, (_TILE, _WIN), 0) >> b) & 1
        x = jnp.where(bit == 1, rolled, x)
    p_ref[0] = x[:, :_LEFT]


def _attn_tc(enc, Mrows, Crows, interpret=False):
    B, S, D = enc.shape
    n_pad = Mrows.shape[0]
    ntiles = S // _TILE
    out, p = pl.pallas_call(
        _attn_body,
        grid=(ntiles,),
        in_specs=[
            pl.BlockSpec((1, _TILE, D), lambda i: (0, i, 0)),
            pl.BlockSpec((128, D), lambda i: (2 * i, 0)),
            pl.BlockSpec((128, D), lambda i: (2 * i + 1, 0)),
            pl.BlockSpec((128, D), lambda i: (2 * i + 2, 0)),
            pl.BlockSpec((128, D), lambda i: (2 * i, 0)),
            pl.BlockSpec((128, D), lambda i: (2 * i + 1, 0)),
            pl.BlockSpec((128, D), lambda i: (2 * i + 2, 0)),
        ],
        out_specs=[
            pl.BlockSpec((1, _TILE, 2 * D), lambda i: (0, i, 0)),
            pl.BlockSpec((1, _TILE, _LEFT), lambda i: (0, i, 0)),
        ],
        out_shape=[
            jax.ShapeDtypeStruct((1, S, 2 * D), jnp.float32),
            jax.ShapeDtypeStruct((1, S, _LEFT), jnp.float32),
        ],
        interpret=interpret,
    )(enc, Mrows, Mrows, Mrows, Crows, Crows, Crows)
    return out, p


def kernel(symbols, encodings, M, C):
    B, S = symbols.shape
    D = encodings.shape[-1]
    sym = symbols[0].astype(jnp.int32)
    n_pad = ((S + _LEFT - 1) // _TILE + 1) * _TILE
    idx_pad = jnp.concatenate([
        jnp.full((_LEFT - 1,), sym[0], jnp.int32),
        sym,
        jnp.zeros((n_pad - S - (_LEFT - 1),), jnp.int32),
    ])
    Mrows, Crows = _sc_gather(M, C, idx_pad)
    return _attn_tc(encodings, Mrows, Crows)
